# Initial kernel scaffold; baseline (speedup 1.0000x reference)
import jax
import jax.numpy as jnp
from jax.experimental import pallas as pl


def kernel(edge_index, h, Wq0, bq0, Wk0, bk0, Wv0, bv0, Wq1, bq1, Wk1, bk1, Wv1, bv1, Wout, bout):
    raise NotImplementedError("write your pallas kernel here")


# speedup vs baseline: 3.2687x; 3.2687x over previous
import math

import jax
import jax.numpy as jnp
from jax import lax
from jax.experimental import pallas as pl
from jax.experimental.pallas import tpu as pltpu
from jax.experimental.pallas import tpu_sc as plsc

N = 10000
E = 320000
D = 128
NCLS = 40
NC, NS = 2, 16
NW = NC * NS
L = 16
NPAD = 10240
NPW = NPAD // NW
NROWS = NPW + 8
SINK = NPW
CH = 96
SCAN = 2000
NSCAN = E // SCAN
CAPP = 12480
QROWS = NPAD + 8
INV_SQRT_D = 1.0 / math.sqrt(D)


def _compact_body(src_hbm, dst_hbm, lists_out, counts_out,
                  src_sc, dst_sc, comp_src, comp_dstl, cbuf):
    c = lax.axis_index("c")
    s_id = lax.axis_index("s")
    wid = s_id * NC + c
    lo = wid * NPW
    iota = lax.iota(jnp.int32, L)

    def scan_chunk(i, cur):
        pltpu.sync_copy(src_hbm.at[pl.ds(i * SCAN, SCAN)], src_sc)
        pltpu.sync_copy(dst_hbm.at[pl.ds(i * SCAN, SCAN)], dst_sc)

        def group(g, cur2):
            sv = src_sc[pl.ds(g * L, L)]
            dv = dst_sc[pl.ds(g * L, L)]
            m = (dv >= lo) & (dv < lo + NPW)
            plsc.store_compressed(comp_src.at[pl.ds(cur2, L)], sv, mask=m)
            plsc.store_compressed(comp_dstl.at[pl.ds(cur2, L)], dv, mask=m)
            return cur2 + jnp.sum(m.astype(jnp.int32), axis=0)

        return lax.fori_loop(0, SCAN // L, group, cur)

    cur = lax.fori_loop(0, NSCAN, scan_chunk, jnp.int32(0))
    for t in range(CH // L):
        plsc.store_scatter(comp_src, [cur + t * L + iota],
                           jnp.zeros((L,), jnp.int32))
        plsc.store_scatter(comp_dstl, [cur + t * L + iota],
                           jnp.full((L,), SINK, jnp.int32) + lo)
    cnt = ((cur + CH - 1) // CH) * CH
    cbuf[...] = jnp.broadcast_to(cnt, (L,))
    pltpu.sync_copy(cbuf, counts_out.at[pl.ds(wid * L, L)])
    pltpu.sync_copy(comp_src, lists_out.at[pl.ds(wid * 2 * CAPP, CAPP)])
    pltpu.sync_copy(comp_dstl, lists_out.at[pl.ds(wid * 2 * CAPP + CAPP, CAPP)])


_compact_call = pl.kernel(
    _compact_body,
    out_type=(jax.ShapeDtypeStruct((NW * 2 * CAPP,), jnp.int32),
              jax.ShapeDtypeStruct((NW * L,), jnp.int32)),
    mesh=plsc.VectorSubcoreMesh(core_axis_name="c", subcore_axis_name="s",
                                num_cores=NC, num_subcores=NS),
    compiler_params=pltpu.CompilerParams(needs_layout_passes=False),
    scratch_types=[
        pltpu.VMEM((SCAN,), jnp.int32),
        pltpu.VMEM((SCAN,), jnp.int32),
        pltpu.VMEM((CAPP,), jnp.int32),
        pltpu.VMEM((CAPP,), jnp.int32),
        pltpu.VMEM((L,), jnp.int32),
    ],
)


def _edge_body(q_hbm, k_hbm, v_hbm, lists_hbm, counts_hbm,
               num_out, den_out,
               src_v, dstl_v, q_rows, k_rows, v_rows, num_loc, den_loc,
               s_buf, cnt_v, sem_q, sem_k, sem_v):
    c = lax.axis_index("c")
    s_id = lax.axis_index("s")
    wid = s_id * NC + c
    lo = wid * NPW
    iota = lax.iota(jnp.int32, L)
    tail = jnp.where(iota == 0, 1.0, 0.0).astype(jnp.float32)
    zero16 = jnp.zeros((L,), jnp.float32)

    def zero_row(r, carry):
        rr = jnp.full((L,), r, jnp.int32)
        for g in range(D // L):
            plsc.store_scatter(num_loc, [rr, g * L + iota], zero16)
        plsc.store_scatter(den_loc, [rr, iota], zero16)
        return carry

    lax.fori_loop(0, NROWS, zero_row, 0)

    pltpu.sync_copy(counts_hbm.at[pl.ds(wid * L, L)], cnt_v)
    nchunks = jnp.max(cnt_v[...], axis=0) // CH
    lbase = wid * 2 * CAPP

    def chunk_body(i, carry):
        o = i * CH
        pltpu.sync_copy(lists_hbm.at[pl.ds(lbase + o, CH)], src_v)
        pltpu.sync_copy(lists_hbm.at[pl.ds(lbase + CAPP + o, CH)], dstl_v)
        cq = pltpu.async_copy(q_hbm.at[dstl_v], q_rows, sem_q)
        ck = pltpu.async_copy(k_hbm.at[src_v], k_rows, sem_k)
        cv = pltpu.async_copy(v_hbm.at[src_v], v_rows, sem_v)
        cq.wait()
        ck.wait()
        cv.wait()

        def sub_body(t, carry2):
            rows16 = t * L + iota

            def dot_body(d, acc):
                dcol = jnp.full((L,), d, jnp.int32)
                qc = plsc.load_gather(q_rows, [rows16, dcol])
                kc = plsc.load_gather(k_rows, [rows16, dcol])
                return acc + qc * kc

            sc = lax.fori_loop(0, D, dot_body, jnp.zeros((L,), jnp.float32),
                               unroll=8)
            s_buf[...] = jnp.exp(sc * INV_SQRT_D)
            for j in range(L):
                jj = jnp.full((L,), j, jnp.int32)
                bs = plsc.load_gather(s_buf, [jj])
                dlv = plsc.load_gather(dstl_v, [jnp.full((L,), t * L + j,
                                                         jnp.int32)]) - lo
                erow = jnp.full((L,), t * L + j, jnp.int32)
                for g in range(D // L):
                    col = g * L + iota
                    vv = plsc.load_gather(v_rows, [erow, col])
                    plsc.addupdate_scatter(num_loc, [dlv, col], bs * vv)
                plsc.addupdate_scatter(den_loc, [dlv, iota], bs * tail)
            return carry2

        lax.fori_loop(0, CH // L, sub_body, 0)
        return carry

    lax.fori_loop(0, nchunks, chunk_body, 0)
    pltpu.sync_copy(num_loc.at[pl.ds(0, NPW), :], num_out.at[pl.ds(lo, NPW), :])
    pltpu.sync_copy(den_loc.at[pl.ds(0, NPW), :], den_out.at[pl.ds(lo, NPW), :])


_edge_call = pl.kernel(
    _edge_body,
    out_type=(jax.ShapeDtypeStruct((NPAD, D), jnp.float32),
              jax.ShapeDtypeStruct((NPAD, L), jnp.float32)),
    mesh=plsc.VectorSubcoreMesh(core_axis_name="c", subcore_axis_name="s",
                                num_cores=NC, num_subcores=NS),
    compiler_params=pltpu.CompilerParams(needs_layout_passes=False),
    scratch_types=[
        pltpu.VMEM((CH,), jnp.int32),
        pltpu.VMEM((CH,), jnp.int32),
        pltpu.VMEM((CH, D), jnp.float32),
        pltpu.VMEM((CH, D), jnp.float32),
        pltpu.VMEM((CH, D), jnp.float32),
        pltpu.VMEM((NROWS, D), jnp.float32),
        pltpu.VMEM((NROWS, L), jnp.float32),
        pltpu.VMEM((L,), jnp.float32),
        pltpu.VMEM((L,), jnp.int32),
        pltpu.SemaphoreType.DMA,
        pltpu.SemaphoreType.DMA,
        pltpu.SemaphoreType.DMA,
    ],
)


BN = 2000
GRID = N // BN


def _qkv_body(h_ref, w_ref, b_ref, q_ref, k_ref, v_ref):
    y = jnp.dot(h_ref[...], w_ref[...],
                preferred_element_type=jnp.float32) + b_ref[...]
    q_ref[...] = y[:, :D]
    k_ref[...] = y[:, D:2 * D]
    v_ref[...] = y[:, 2 * D:]


def _qkv_call(h, w, b):
    return pl.pallas_call(
        _qkv_body,
        grid=(GRID,),
        in_specs=[
            pl.BlockSpec((BN, D), lambda i: (i, 0)),
            pl.BlockSpec((D, 3 * D), lambda i: (0, 0)),
            pl.BlockSpec((1, 3 * D), lambda i: (0, 0)),
        ],
        out_specs=(pl.BlockSpec((BN, D), lambda i: (i, 0)),
                   pl.BlockSpec((BN, D), lambda i: (i, 0)),
                   pl.BlockSpec((BN, D), lambda i: (i, 0))),
        out_shape=(jax.ShapeDtypeStruct((N, D), jnp.float32),
                   jax.ShapeDtypeStruct((N, D), jnp.float32),
                   jax.ShapeDtypeStruct((N, D), jnp.float32)),
    )(h, w, b)


def _normalize(num_ref, den_ref):
    return num_ref[...] / (den_ref[...][:, :1] + 1e-9)


def _combine_body(num_ref, den_ref, w_ref, b_ref, q_ref, k_ref, v_ref):
    h = _normalize(num_ref, den_ref)
    y = jnp.dot(h, w_ref[...], preferred_element_type=jnp.float32) + b_ref[...]
    q_ref[...] = y[:, :D]
    k_ref[...] = y[:, D:2 * D]
    v_ref[...] = y[:, 2 * D:]


def _combine_call(num, den, w, b):
    return pl.pallas_call(
        _combine_body,
        grid=(GRID,),
        in_specs=[
            pl.BlockSpec((BN, D), lambda i: (i, 0)),
            pl.BlockSpec((BN, L), lambda i: (i, 0)),
            pl.BlockSpec((D, 3 * D), lambda i: (0, 0)),
            pl.BlockSpec((1, 3 * D), lambda i: (0, 0)),
        ],
        out_specs=(pl.BlockSpec((BN, D), lambda i: (i, 0)),
                   pl.BlockSpec((BN, D), lambda i: (i, 0)),
                   pl.BlockSpec((BN, D), lambda i: (i, 0))),
        out_shape=(jax.ShapeDtypeStruct((N, D), jnp.float32),
                   jax.ShapeDtypeStruct((N, D), jnp.float32),
                   jax.ShapeDtypeStruct((N, D), jnp.float32)),
    )(num, den, w, b)


def _final_body(num_ref, den_ref, w_ref, b_ref, out_ref):
    h = _normalize(num_ref, den_ref)
    logits = jnp.dot(h, w_ref[...],
                     preferred_element_type=jnp.float32) + b_ref[...]
    m = jnp.max(logits, axis=1, keepdims=True)
    z = logits - m
    lse = jnp.log(jnp.sum(jnp.exp(z), axis=1, keepdims=True))
    out_ref[...] = z - lse


def _final_call(num, den, w, b):
    return pl.pallas_call(
        _final_body,
        grid=(GRID,),
        in_specs=[
            pl.BlockSpec((BN, D), lambda i: (i, 0)),
            pl.BlockSpec((BN, L), lambda i: (i, 0)),
            pl.BlockSpec((D, NCLS), lambda i: (0, 0)),
            pl.BlockSpec((1, NCLS), lambda i: (0, 0)),
        ],
        out_specs=pl.BlockSpec((BN, NCLS), lambda i: (i, 0)),
        out_shape=jax.ShapeDtypeStruct((N, NCLS), jnp.float32),
    )(num, den, w, b)


def kernel(edge_index, h, Wq0, bq0, Wk0, bk0, Wv0, bv0,
           Wq1, bq1, Wk1, bk1, Wv1, bv1, Wout, bout):
    src = edge_index[0].astype(jnp.int32)
    dst = edge_index[1].astype(jnp.int32)
    W0 = jnp.concatenate([Wq0, Wk0, Wv0], axis=1)
    b0 = jnp.concatenate([bq0, bk0, bv0])[None, :]
    W1 = jnp.concatenate([Wq1, Wk1, Wv1], axis=1)
    b1 = jnp.concatenate([bq1, bk1, bv1])[None, :]

    lists, counts = _compact_call(src, dst)

    q0, k0, v0 = _qkv_call(h, W0, b0)
    q0p = jnp.pad(q0, ((0, QROWS - N), (0, 0)))
    num0, den0 = _edge_call(q0p, k0, v0, lists, counts)
    q1, k1, v1 = _combine_call(num0[:N], den0[:N], W1, b1)
    q1p = jnp.pad(q1, ((0, QROWS - N), (0, 0)))
    num1, den1 = _edge_call(q1p, k1, v1, lists, counts)
    return _final_call(num1[:N], den1[:N], Wout, bout[None, :])


# speedup vs baseline: 3.7063x; 1.1339x over previous
import math

import jax
import jax.numpy as jnp
from jax import lax
from jax.experimental import pallas as pl
from jax.experimental.pallas import tpu as pltpu
from jax.experimental.pallas import tpu_sc as plsc

N = 10000
E = 320000
D = 128
NCLS = 40
NC, NS = 2, 16
NW = NC * NS
L = 16
NPAD = 10240
NPW = NPAD // NW
NROWS = NPW + 8
SINK = NPW
CH = 48
SCAN = 4000
NSCAN = E // SCAN
CAPP = 12480
QROWS = NPAD + 8
INV_SQRT_D = 1.0 / math.sqrt(D)


def _compact_body(src_hbm, dst_hbm, lists_out, counts_out,
                  src_a, dst_a, src_b, dst_b, comp_src, comp_dstl, cbuf,
                  sem_a, sem_b):
    c = lax.axis_index("c")
    s_id = lax.axis_index("s")
    wid = s_id * NC + c
    lo = wid * NPW
    iota = lax.iota(jnp.int32, L)

    def issue(sbuf, dbuf, i, sem):
        o = jnp.minimum(i, NSCAN - 1) * SCAN
        pltpu.async_copy(src_hbm.at[pl.ds(o, SCAN)], sbuf, sem)
        pltpu.async_copy(dst_hbm.at[pl.ds(o, SCAN)], dbuf, sem)

    def drain(sbuf, dbuf, sem):
        pltpu.make_async_copy(src_hbm.at[pl.ds(0, SCAN)], sbuf, sem).wait()
        pltpu.make_async_copy(dst_hbm.at[pl.ds(0, SCAN)], dbuf, sem).wait()

    def process(sbuf, dbuf, cur):
        def group(g, cur2):
            sv = sbuf[pl.ds(g * L, L)]
            dv = dbuf[pl.ds(g * L, L)]
            m = (dv >= lo) & (dv < lo + NPW)
            plsc.store_compressed(comp_src.at[pl.ds(cur2, L)], sv, mask=m)
            plsc.store_compressed(comp_dstl.at[pl.ds(cur2, L)], dv, mask=m)
            return cur2 + jnp.sum(m.astype(jnp.int32), axis=0)

        return lax.fori_loop(0, SCAN // L, group, cur)

    issue(src_a, dst_a, 0, sem_a)

    def scan2(i2, cur):
        drain(src_a, dst_a, sem_a)
        issue(src_b, dst_b, 2 * i2 + 1, sem_b)
        cur = process(src_a, dst_a, cur)
        drain(src_b, dst_b, sem_b)
        issue(src_a, dst_a, 2 * i2 + 2, sem_a)
        return process(src_b, dst_b, cur)

    cur = lax.fori_loop(0, NSCAN // 2, scan2, jnp.int32(0))
    drain(src_a, dst_a, sem_a)
    for t in range(2 * CH // L):
        plsc.store_scatter(comp_src, [cur + t * L + iota],
                           jnp.zeros((L,), jnp.int32))
        plsc.store_scatter(comp_dstl, [cur + t * L + iota],
                           jnp.full((L,), SINK, jnp.int32) + lo)
    cnt = ((cur + 2 * CH - 1) // (2 * CH)) * (2 * CH)
    cbuf[...] = jnp.broadcast_to(cnt, (L,))
    pltpu.sync_copy(cbuf, counts_out.at[pl.ds(wid * L, L)])
    pltpu.sync_copy(comp_src, lists_out.at[pl.ds(wid * 2 * CAPP, CAPP)])
    pltpu.sync_copy(comp_dstl, lists_out.at[pl.ds(wid * 2 * CAPP + CAPP, CAPP)])


_compact_call = pl.kernel(
    _compact_body,
    out_type=(jax.ShapeDtypeStruct((NW * 2 * CAPP,), jnp.int32),
              jax.ShapeDtypeStruct((NW * L,), jnp.int32)),
    mesh=plsc.VectorSubcoreMesh(core_axis_name="c", subcore_axis_name="s",
                                num_cores=NC, num_subcores=NS),
    compiler_params=pltpu.CompilerParams(needs_layout_passes=False),
    scratch_types=[
        pltpu.VMEM((SCAN,), jnp.int32),
        pltpu.VMEM((SCAN,), jnp.int32),
        pltpu.VMEM((SCAN,), jnp.int32),
        pltpu.VMEM((SCAN,), jnp.int32),
        pltpu.VMEM((CAPP,), jnp.int32),
        pltpu.VMEM((CAPP,), jnp.int32),
        pltpu.VMEM((L,), jnp.int32),
        pltpu.SemaphoreType.DMA,
        pltpu.SemaphoreType.DMA,
    ],
)


def _edge_body(q_hbm, k_hbm, v_hbm, lists_hbm, counts_hbm,
               num_out, den_out,
               src_a, dstl_a, q_a, k_a, v_a,
               src_b, dstl_b, q_b, k_b, v_b,
               num_loc, den_loc, s_buf, cnt_v,
               sem_ia, sem_ib, sem_ga, sem_gb):
    c = lax.axis_index("c")
    s_id = lax.axis_index("s")
    wid = s_id * NC + c
    lo = wid * NPW
    iota = lax.iota(jnp.int32, L)
    tail = jnp.where(iota == 0, 1.0, 0.0).astype(jnp.float32)
    zero16 = jnp.zeros((L,), jnp.float32)

    def zero_row(r, carry):
        rr = jnp.full((L,), r, jnp.int32)
        for g in range(D // L):
            plsc.store_scatter(num_loc, [rr, g * L + iota], zero16)
        plsc.store_scatter(den_loc, [rr, iota], zero16)
        return carry

    lax.fori_loop(0, NROWS, zero_row, 0)

    pltpu.sync_copy(counts_hbm.at[pl.ds(wid * L, L)], cnt_v)
    nchunks = jnp.max(cnt_v[...], axis=0) // CH
    lbase = wid * 2 * CAPP

    def issue_idx(sbuf, dbuf, ci, sem):
        o = lbase + jnp.minimum(ci, nchunks - 1) * CH
        pltpu.async_copy(lists_hbm.at[pl.ds(o, CH)], sbuf, sem)
        pltpu.async_copy(lists_hbm.at[pl.ds(o + CAPP, CH)], dbuf, sem)

    def drain_idx(sbuf, dbuf, sem):
        pltpu.make_async_copy(lists_hbm.at[pl.ds(lbase, CH)], sbuf, sem).wait()
        pltpu.make_async_copy(lists_hbm.at[pl.ds(lbase, CH)], dbuf, sem).wait()

    def issue_g(sbuf, dbuf, qb, kb, vb, sem):
        pltpu.async_copy(q_hbm.at[dbuf], qb, sem)
        pltpu.async_copy(k_hbm.at[sbuf], kb, sem)
        pltpu.async_copy(v_hbm.at[sbuf], vb, sem)

    def drain_g(sbuf, dbuf, qb, kb, vb, sem):
        pltpu.make_async_copy(q_hbm.at[dbuf], qb, sem).wait()
        pltpu.make_async_copy(k_hbm.at[sbuf], kb, sem).wait()
        pltpu.make_async_copy(v_hbm.at[sbuf], vb, sem).wait()

    def compute(dbuf, qb, kb, vb):
        def sub_body(t, carry2):
            rows16 = t * L + iota

            def dot_body(d, acc):
                dcol = jnp.full((L,), d, jnp.int32)
                qc = plsc.load_gather(qb, [rows16, dcol])
                kc = plsc.load_gather(kb, [rows16, dcol])
                return acc + qc * kc

            sc = lax.fori_loop(0, D, dot_body, jnp.zeros((L,), jnp.float32),
                               unroll=8)
            s_buf[...] = jnp.exp(sc * INV_SQRT_D)
            for j in range(L):
                jj = jnp.full((L,), j, jnp.int32)
                bs = plsc.load_gather(s_buf, [jj])
                dlv = plsc.load_gather(dbuf, [jnp.full((L,), t * L + j,
                                                       jnp.int32)]) - lo
                erow = jnp.full((L,), t * L + j, jnp.int32)
                for g in range(D // L):
                    col = g * L + iota
                    vv = plsc.load_gather(vb, [erow, col])
                    plsc.addupdate_scatter(num_loc, [dlv, col], bs * vv)
                plsc.addupdate_scatter(den_loc, [dlv, iota], bs * tail)
            return carry2

        lax.fori_loop(0, CH // L, sub_body, 0)

    issue_idx(src_a, dstl_a, 0, sem_ia)
    drain_idx(src_a, dstl_a, sem_ia)
    issue_g(src_a, dstl_a, q_a, k_a, v_a, sem_ga)
    issue_idx(src_b, dstl_b, 1, sem_ib)

    def chunk2(i2, carry):
        drain_idx(src_b, dstl_b, sem_ib)
        issue_g(src_b, dstl_b, q_b, k_b, v_b, sem_gb)
        drain_g(src_a, dstl_a, q_a, k_a, v_a, sem_ga)
        compute(dstl_a, q_a, k_a, v_a)
        issue_idx(src_a, dstl_a, 2 * i2 + 2, sem_ia)
        drain_idx(src_a, dstl_a, sem_ia)
        issue_g(src_a, dstl_a, q_a, k_a, v_a, sem_ga)
        drain_g(src_b, dstl_b, q_b, k_b, v_b, sem_gb)
        compute(dstl_b, q_b, k_b, v_b)
        issue_idx(src_b, dstl_b, 2 * i2 + 3, sem_ib)
        return carry

    lax.fori_loop(0, nchunks // 2, chunk2, 0)
    drain_g(src_a, dstl_a, q_a, k_a, v_a, sem_ga)
    drain_idx(src_b, dstl_b, sem_ib)
    pltpu.sync_copy(num_loc.at[pl.ds(0, NPW), :], num_out.at[pl.ds(lo, NPW), :])
    pltpu.sync_copy(den_loc.at[pl.ds(0, NPW), :], den_out.at[pl.ds(lo, NPW), :])


_edge_call = pl.kernel(
    _edge_body,
    out_type=(jax.ShapeDtypeStruct((NPAD, D), jnp.float32),
              jax.ShapeDtypeStruct((NPAD, L), jnp.float32)),
    mesh=plsc.VectorSubcoreMesh(core_axis_name="c", subcore_axis_name="s",
                                num_cores=NC, num_subcores=NS),
    compiler_params=pltpu.CompilerParams(needs_layout_passes=False),
    scratch_types=[
        pltpu.VMEM((CH,), jnp.int32),
        pltpu.VMEM((CH,), jnp.int32),
        pltpu.VMEM((CH, D), jnp.float32),
        pltpu.VMEM((CH, D), jnp.float32),
        pltpu.VMEM((CH, D), jnp.float32),
        pltpu.VMEM((CH,), jnp.int32),
        pltpu.VMEM((CH,), jnp.int32),
        pltpu.VMEM((CH, D), jnp.float32),
        pltpu.VMEM((CH, D), jnp.float32),
        pltpu.VMEM((CH, D), jnp.float32),
        pltpu.VMEM((NROWS, D), jnp.float32),
        pltpu.VMEM((NROWS, L), jnp.float32),
        pltpu.VMEM((L,), jnp.float32),
        pltpu.VMEM((L,), jnp.int32),
        pltpu.SemaphoreType.DMA,
        pltpu.SemaphoreType.DMA,
        pltpu.SemaphoreType.DMA,
        pltpu.SemaphoreType.DMA,
    ],
)


BN = 2000
GRID = N // BN


def _qkv_body(h_ref, w_ref, b_ref, q_ref, k_ref, v_ref):
    y = jnp.dot(h_ref[...], w_ref[...],
                preferred_element_type=jnp.float32) + b_ref[...]
    q_ref[...] = y[:, :D]
    k_ref[...] = y[:, D:2 * D]
    v_ref[...] = y[:, 2 * D:]


def _qkv_call(h, w, b):
    return pl.pallas_call(
        _qkv_body,
        grid=(GRID,),
        in_specs=[
            pl.BlockSpec((BN, D), lambda i: (i, 0)),
            pl.BlockSpec((D, 3 * D), lambda i: (0, 0)),
            pl.BlockSpec((1, 3 * D), lambda i: (0, 0)),
        ],
        out_specs=(pl.BlockSpec((BN, D), lambda i: (i, 0)),
                   pl.BlockSpec((BN, D), lambda i: (i, 0)),
                   pl.BlockSpec((BN, D), lambda i: (i, 0))),
        out_shape=(jax.ShapeDtypeStruct((N, D), jnp.float32),
                   jax.ShapeDtypeStruct((N, D), jnp.float32),
                   jax.ShapeDtypeStruct((N, D), jnp.float32)),
    )(h, w, b)


def _normalize(num_ref, den_ref):
    return num_ref[...] / (den_ref[...][:, :1] + 1e-9)


def _combine_body(num_ref, den_ref, w_ref, b_ref, q_ref, k_ref, v_ref):
    h = _normalize(num_ref, den_ref)
    y = jnp.dot(h, w_ref[...], preferred_element_type=jnp.float32) + b_ref[...]
    q_ref[...] = y[:, :D]
    k_ref[...] = y[:, D:2 * D]
    v_ref[...] = y[:, 2 * D:]


def _combine_call(num, den, w, b):
    return pl.pallas_call(
        _combine_body,
        grid=(GRID,),
        in_specs=[
            pl.BlockSpec((BN, D), lambda i: (i, 0)),
            pl.BlockSpec((BN, L), lambda i: (i, 0)),
            pl.BlockSpec((D, 3 * D), lambda i: (0, 0)),
            pl.BlockSpec((1, 3 * D), lambda i: (0, 0)),
        ],
        out_specs=(pl.BlockSpec((BN, D), lambda i: (i, 0)),
                   pl.BlockSpec((BN, D), lambda i: (i, 0)),
                   pl.BlockSpec((BN, D), lambda i: (i, 0))),
        out_shape=(jax.ShapeDtypeStruct((N, D), jnp.float32),
                   jax.ShapeDtypeStruct((N, D), jnp.float32),
                   jax.ShapeDtypeStruct((N, D), jnp.float32)),
    )(num, den, w, b)


def _final_body(num_ref, den_ref, w_ref, b_ref, out_ref):
    h = _normalize(num_ref, den_ref)
    logits = jnp.dot(h, w_ref[...],
                     preferred_element_type=jnp.float32) + b_ref[...]
    m = jnp.max(logits, axis=1, keepdims=True)
    z = logits - m
    lse = jnp.log(jnp.sum(jnp.exp(z), axis=1, keepdims=True))
    out_ref[...] = z - lse


def _final_call(num, den, w, b):
    return pl.pallas_call(
        _final_body,
        grid=(GRID,),
        in_specs=[
            pl.BlockSpec((BN, D), lambda i: (i, 0)),
            pl.BlockSpec((BN, L), lambda i: (i, 0)),
            pl.BlockSpec((D, NCLS), lambda i: (0, 0)),
            pl.BlockSpec((1, NCLS), lambda i: (0, 0)),
        ],
        out_specs=pl.BlockSpec((BN, NCLS), lambda i: (i, 0)),
        out_shape=jax.ShapeDtypeStruct((N, NCLS), jnp.float32),
    )(num, den, w, b)


def kernel(edge_index, h, Wq0, bq0, Wk0, bk0, Wv0, bv0,
           Wq1, bq1, Wk1, bk1, Wv1, bv1, Wout, bout):
    src = edge_index[0].astype(jnp.int32)
    dst = edge_index[1].astype(jnp.int32)
    W0 = jnp.concatenate([Wq0, Wk0, Wv0], axis=1)
    b0 = jnp.concatenate([bq0, bk0, bv0])[None, :]
    W1 = jnp.concatenate([Wq1, Wk1, Wv1], axis=1)
    b1 = jnp.concatenate([bq1, bk1, bv1])[None, :]

    lists, counts = _compact_call(src, dst)

    q0, k0, v0 = _qkv_call(h, W0, b0)
    q0p = jnp.pad(q0, ((0, QROWS - N), (0, 0)))
    num0, den0 = _edge_call(q0p, k0, v0, lists, counts)
    q1, k1, v1 = _combine_call(num0[:N], den0[:N], W1, b1)
    q1p = jnp.pad(q1, ((0, QROWS - N), (0, 0)))
    num1, den1 = _edge_call(q1p, k1, v1, lists, counts)
    return _final_call(num1[:N], den1[:N], Wout, bout[None, :])


# speedup vs baseline: 3.9921x; 1.0771x over previous
import math

import jax
import jax.numpy as jnp
from jax import lax
from jax.experimental import pallas as pl
from jax.experimental.pallas import tpu as pltpu
from jax.experimental.pallas import tpu_sc as plsc

N = 10000
E = 320000
D = 128
NCLS = 40
NC, NS = 2, 16
NW = NC * NS
L = 16
NPAD = 10240
NPW = NPAD // NW
NROWS = NPW + 8
SINK = NPW
CH = 48
SCAN = 4000
NSCAN = E // SCAN
CAPP = 12480
QROWS = NPAD + 8
INV_SQRT_D = 1.0 / math.sqrt(D)


def _compact_body(src_hbm, dst_hbm, lists_out, counts_out,
                  src_a, dst_a, src_b, dst_b, comp_src, comp_dstl, cbuf,
                  sem_a, sem_b):
    c = lax.axis_index("c")
    s_id = lax.axis_index("s")
    wid = s_id * NC + c
    lo = wid * NPW
    iota = lax.iota(jnp.int32, L)

    def issue(sbuf, dbuf, i, sem):
        o = jnp.minimum(i, NSCAN - 1) * SCAN
        pltpu.async_copy(src_hbm.at[pl.ds(o, SCAN)], sbuf, sem)
        pltpu.async_copy(dst_hbm.at[pl.ds(o, SCAN)], dbuf, sem)

    def drain(sbuf, dbuf, sem):
        pltpu.make_async_copy(src_hbm.at[pl.ds(0, SCAN)], sbuf, sem).wait()
        pltpu.make_async_copy(dst_hbm.at[pl.ds(0, SCAN)], dbuf, sem).wait()

    def process(sbuf, dbuf, cur):
        def group(g, cur2):
            sv = sbuf[pl.ds(g * L, L)]
            dv = dbuf[pl.ds(g * L, L)]
            m = (dv >= lo) & (dv < lo + NPW)
            plsc.store_compressed(comp_src.at[pl.ds(cur2, L)], sv, mask=m)
            plsc.store_compressed(comp_dstl.at[pl.ds(cur2, L)], dv, mask=m)
            return cur2 + jnp.sum(m.astype(jnp.int32), axis=0)

        return lax.fori_loop(0, SCAN // L, group, cur)

    issue(src_a, dst_a, 0, sem_a)

    def scan2(i2, cur):
        drain(src_a, dst_a, sem_a)
        issue(src_b, dst_b, 2 * i2 + 1, sem_b)
        cur = process(src_a, dst_a, cur)
        drain(src_b, dst_b, sem_b)
        issue(src_a, dst_a, 2 * i2 + 2, sem_a)
        return process(src_b, dst_b, cur)

    cur = lax.fori_loop(0, NSCAN // 2, scan2, jnp.int32(0))
    drain(src_a, dst_a, sem_a)
    for t in range(2 * CH // L):
        plsc.store_scatter(comp_src, [cur + t * L + iota],
                           jnp.zeros((L,), jnp.int32))
        plsc.store_scatter(comp_dstl, [cur + t * L + iota],
                           jnp.full((L,), SINK, jnp.int32) + lo)
    cnt = ((cur + 2 * CH - 1) // (2 * CH)) * (2 * CH)
    cbuf[...] = jnp.broadcast_to(cnt, (L,))
    pltpu.sync_copy(cbuf, counts_out.at[pl.ds(wid * L, L)])
    pltpu.sync_copy(comp_src, lists_out.at[pl.ds(wid * 2 * CAPP, CAPP)])
    pltpu.sync_copy(comp_dstl, lists_out.at[pl.ds(wid * 2 * CAPP + CAPP, CAPP)])


_compact_call = pl.kernel(
    _compact_body,
    out_type=(jax.ShapeDtypeStruct((NW * 2 * CAPP,), jnp.int32),
              jax.ShapeDtypeStruct((NW * L,), jnp.int32)),
    mesh=plsc.VectorSubcoreMesh(core_axis_name="c", subcore_axis_name="s",
                                num_cores=NC, num_subcores=NS),
    compiler_params=pltpu.CompilerParams(needs_layout_passes=False),
    scratch_types=[
        pltpu.VMEM((SCAN,), jnp.int32),
        pltpu.VMEM((SCAN,), jnp.int32),
        pltpu.VMEM((SCAN,), jnp.int32),
        pltpu.VMEM((SCAN,), jnp.int32),
        pltpu.VMEM((CAPP,), jnp.int32),
        pltpu.VMEM((CAPP,), jnp.int32),
        pltpu.VMEM((L,), jnp.int32),
        pltpu.SemaphoreType.DMA,
        pltpu.SemaphoreType.DMA,
    ],
)


def _edge_body(q_hbm, k_hbm, v_hbm, lists_hbm, counts_hbm,
               num_out, den_out,
               src_a, dstl_a, q_a, k_a, v_a,
               src_b, dstl_b, q_b, k_b, v_b,
               num_loc, den_loc, s_buf, cnt_v,
               sem_ia, sem_ib, sem_ga, sem_gb):
    c = lax.axis_index("c")
    s_id = lax.axis_index("s")
    wid = s_id * NC + c
    lo = wid * NPW
    iota = lax.iota(jnp.int32, L)
    tail = jnp.where(iota == 0, 1.0, 0.0).astype(jnp.float32)
    zero16 = jnp.zeros((L,), jnp.float32)

    def zero_row(r, carry):
        rr = jnp.full((L,), r, jnp.int32)
        for g in range(D // L):
            plsc.store_scatter(num_loc, [rr, g * L + iota], zero16)
        plsc.store_scatter(den_loc, [rr, iota], zero16)
        return carry

    lax.fori_loop(0, NROWS, zero_row, 0)

    pltpu.sync_copy(counts_hbm.at[pl.ds(wid * L, L)], cnt_v)
    nchunks = jnp.max(cnt_v[...], axis=0) // CH
    lbase = wid * 2 * CAPP

    def issue_idx(sbuf, dbuf, ci, sem):
        o = lbase + jnp.minimum(ci, nchunks - 1) * CH
        pltpu.async_copy(lists_hbm.at[pl.ds(o, CH)], sbuf, sem)
        pltpu.async_copy(lists_hbm.at[pl.ds(o + CAPP, CH)], dbuf, sem)

    def drain_idx(sbuf, dbuf, sem):
        pltpu.make_async_copy(lists_hbm.at[pl.ds(lbase, CH)], sbuf, sem).wait()
        pltpu.make_async_copy(lists_hbm.at[pl.ds(lbase, CH)], dbuf, sem).wait()

    def issue_g(sbuf, dbuf, qb, kb, vb, sem):
        pltpu.async_copy(q_hbm.at[dbuf], qb, sem)
        pltpu.async_copy(k_hbm.at[sbuf], kb, sem)
        pltpu.async_copy(v_hbm.at[sbuf], vb, sem)

    def drain_g(sbuf, dbuf, qb, kb, vb, sem):
        pltpu.make_async_copy(q_hbm.at[dbuf], qb, sem).wait()
        pltpu.make_async_copy(k_hbm.at[sbuf], kb, sem).wait()
        pltpu.make_async_copy(v_hbm.at[sbuf], vb, sem).wait()

    def compute(dbuf, qb, kb, vb):
        def sub_body(t, carry2):
            rows16 = t * L + iota
            zacc = jnp.zeros((L,), jnp.float32)

            def dot_body(i, accs):
                a0, a1, a2, a3 = accs
                acc4 = [a0, a1, a2, a3]
                for dd in range(8):
                    dcol = jnp.full((L,), i * 8 + dd, jnp.int32)
                    qc = plsc.load_gather(qb, [rows16, dcol])
                    kc = plsc.load_gather(kb, [rows16, dcol])
                    acc4[dd % 4] = acc4[dd % 4] + qc * kc
                return tuple(acc4)

            a0, a1, a2, a3 = lax.fori_loop(0, D // 8, dot_body,
                                           (zacc, zacc, zacc, zacc))
            sc = (a0 + a1) + (a2 + a3)
            s_buf[...] = jnp.exp(sc * INV_SQRT_D)

            def j_body(j, carry3):
                jj = jnp.full((L,), j, jnp.int32)
                bs = plsc.load_gather(s_buf, [jj])
                erow = jnp.full((L,), t * L + j, jnp.int32)
                dlv = plsc.load_gather(dbuf, [erow]) - lo
                for g in range(D // L):
                    col = g * L + iota
                    vv = plsc.load_gather(vb, [erow, col])
                    plsc.addupdate_scatter(num_loc, [dlv, col], bs * vv)
                plsc.addupdate_scatter(den_loc, [dlv, iota], bs * tail)
                return carry3

            lax.fori_loop(0, L, j_body, 0)
            return carry2

        lax.fori_loop(0, CH // L, sub_body, 0)

    issue_idx(src_a, dstl_a, 0, sem_ia)
    drain_idx(src_a, dstl_a, sem_ia)
    issue_g(src_a, dstl_a, q_a, k_a, v_a, sem_ga)
    issue_idx(src_b, dstl_b, 1, sem_ib)

    def chunk2(i2, carry):
        drain_idx(src_b, dstl_b, sem_ib)
        issue_g(src_b, dstl_b, q_b, k_b, v_b, sem_gb)
        drain_g(src_a, dstl_a, q_a, k_a, v_a, sem_ga)
        compute(dstl_a, q_a, k_a, v_a)
        issue_idx(src_a, dstl_a, 2 * i2 + 2, sem_ia)
        drain_idx(src_a, dstl_a, sem_ia)
        issue_g(src_a, dstl_a, q_a, k_a, v_a, sem_ga)
        drain_g(src_b, dstl_b, q_b, k_b, v_b, sem_gb)
        compute(dstl_b, q_b, k_b, v_b)
        issue_idx(src_b, dstl_b, 2 * i2 + 3, sem_ib)
        return carry

    lax.fori_loop(0, nchunks // 2, chunk2, 0)
    drain_g(src_a, dstl_a, q_a, k_a, v_a, sem_ga)
    drain_idx(src_b, dstl_b, sem_ib)
    pltpu.sync_copy(num_loc.at[pl.ds(0, NPW), :], num_out.at[pl.ds(lo, NPW), :])
    pltpu.sync_copy(den_loc.at[pl.ds(0, NPW), :], den_out.at[pl.ds(lo, NPW), :])


_edge_call = pl.kernel(
    _edge_body,
    out_type=(jax.ShapeDtypeStruct((NPAD, D), jnp.float32),
              jax.ShapeDtypeStruct((NPAD, L), jnp.float32)),
    mesh=plsc.VectorSubcoreMesh(core_axis_name="c", subcore_axis_name="s",
                                num_cores=NC, num_subcores=NS),
    compiler_params=pltpu.CompilerParams(needs_layout_passes=False),
    scratch_types=[
        pltpu.VMEM((CH,), jnp.int32),
        pltpu.VMEM((CH,), jnp.int32),
        pltpu.VMEM((CH, D), jnp.float32),
        pltpu.VMEM((CH, D), jnp.float32),
        pltpu.VMEM((CH, D), jnp.float32),
        pltpu.VMEM((CH,), jnp.int32),
        pltpu.VMEM((CH,), jnp.int32),
        pltpu.VMEM((CH, D), jnp.float32),
        pltpu.VMEM((CH, D), jnp.float32),
        pltpu.VMEM((CH, D), jnp.float32),
        pltpu.VMEM((NROWS, D), jnp.float32),
        pltpu.VMEM((NROWS, L), jnp.float32),
        pltpu.VMEM((L,), jnp.float32),
        pltpu.VMEM((L,), jnp.int32),
        pltpu.SemaphoreType.DMA,
        pltpu.SemaphoreType.DMA,
        pltpu.SemaphoreType.DMA,
        pltpu.SemaphoreType.DMA,
    ],
)


BN = 2000
GRID = N // BN


def _qkv_body(h_ref, w_ref, b_ref, q_ref, k_ref, v_ref):
    y = jnp.dot(h_ref[...], w_ref[...],
                preferred_element_type=jnp.float32) + b_ref[...]
    q_ref[...] = y[:, :D]
    k_ref[...] = y[:, D:2 * D]
    v_ref[...] = y[:, 2 * D:]


def _qkv_call(h, w, b):
    return pl.pallas_call(
        _qkv_body,
        grid=(GRID,),
        in_specs=[
            pl.BlockSpec((BN, D), lambda i: (i, 0)),
            pl.BlockSpec((D, 3 * D), lambda i: (0, 0)),
            pl.BlockSpec((1, 3 * D), lambda i: (0, 0)),
        ],
        out_specs=(pl.BlockSpec((BN, D), lambda i: (i, 0)),
                   pl.BlockSpec((BN, D), lambda i: (i, 0)),
                   pl.BlockSpec((BN, D), lambda i: (i, 0))),
        out_shape=(jax.ShapeDtypeStruct((N, D), jnp.float32),
                   jax.ShapeDtypeStruct((N, D), jnp.float32),
                   jax.ShapeDtypeStruct((N, D), jnp.float32)),
    )(h, w, b)


def _normalize(num_ref, den_ref):
    return num_ref[...] / (den_ref[...][:, :1] + 1e-9)


def _combine_body(num_ref, den_ref, w_ref, b_ref, q_ref, k_ref, v_ref):
    h = _normalize(num_ref, den_ref)
    y = jnp.dot(h, w_ref[...], preferred_element_type=jnp.float32) + b_ref[...]
    q_ref[...] = y[:, :D]
    k_ref[...] = y[:, D:2 * D]
    v_ref[...] = y[:, 2 * D:]


def _combine_call(num, den, w, b):
    return pl.pallas_call(
        _combine_body,
        grid=(GRID,),
        in_specs=[
            pl.BlockSpec((BN, D), lambda i: (i, 0)),
            pl.BlockSpec((BN, L), lambda i: (i, 0)),
            pl.BlockSpec((D, 3 * D), lambda i: (0, 0)),
            pl.BlockSpec((1, 3 * D), lambda i: (0, 0)),
        ],
        out_specs=(pl.BlockSpec((BN, D), lambda i: (i, 0)),
                   pl.BlockSpec((BN, D), lambda i: (i, 0)),
                   pl.BlockSpec((BN, D), lambda i: (i, 0))),
        out_shape=(jax.ShapeDtypeStruct((N, D), jnp.float32),
                   jax.ShapeDtypeStruct((N, D), jnp.float32),
                   jax.ShapeDtypeStruct((N, D), jnp.float32)),
    )(num, den, w, b)


def _final_body(num_ref, den_ref, w_ref, b_ref, out_ref):
    h = _normalize(num_ref, den_ref)
    logits = jnp.dot(h, w_ref[...],
                     preferred_element_type=jnp.float32) + b_ref[...]
    m = jnp.max(logits, axis=1, keepdims=True)
    z = logits - m
    lse = jnp.log(jnp.sum(jnp.exp(z), axis=1, keepdims=True))
    out_ref[...] = z - lse


def _final_call(num, den, w, b):
    return pl.pallas_call(
        _final_body,
        grid=(GRID,),
        in_specs=[
            pl.BlockSpec((BN, D), lambda i: (i, 0)),
            pl.BlockSpec((BN, L), lambda i: (i, 0)),
            pl.BlockSpec((D, NCLS), lambda i: (0, 0)),
            pl.BlockSpec((1, NCLS), lambda i: (0, 0)),
        ],
        out_specs=pl.BlockSpec((BN, NCLS), lambda i: (i, 0)),
        out_shape=jax.ShapeDtypeStruct((N, NCLS), jnp.float32),
    )(num, den, w, b)


def kernel(edge_index, h, Wq0, bq0, Wk0, bk0, Wv0, bv0,
           Wq1, bq1, Wk1, bk1, Wv1, bv1, Wout, bout):
    src = edge_index[0].astype(jnp.int32)
    dst = edge_index[1].astype(jnp.int32)
    W0 = jnp.concatenate([Wq0, Wk0, Wv0], axis=1)
    b0 = jnp.concatenate([bq0, bk0, bv0])[None, :]
    W1 = jnp.concatenate([Wq1, Wk1, Wv1], axis=1)
    b1 = jnp.concatenate([bq1, bk1, bv1])[None, :]

    lists, counts = _compact_call(src, dst)

    q0, k0, v0 = _qkv_call(h, W0, b0)
    q0p = jnp.pad(q0, ((0, QROWS - N), (0, 0)))
    num0, den0 = _edge_call(q0p, k0, v0, lists, counts)
    q1, k1, v1 = _combine_call(num0[:N], den0[:N], W1, b1)
    q1p = jnp.pad(q1, ((0, QROWS - N), (0, 0)))
    num1, den1 = _edge_call(q1p, k1, v1, lists, counts)
    return _final_call(num1[:N], den1[:N], Wout, bout[None, :])
